# Initial kernel scaffold; baseline (speedup 1.0000x reference)
import jax
import jax.numpy as jnp
from jax.experimental import pallas as pl


def kernel(boxes, pos_weight):
    raise NotImplementedError("write your pallas kernel here")


# speedup vs baseline: 1.5652x; 1.5652x over previous
import functools

import jax
import jax.numpy as jnp
from jax import lax
from jax.experimental import pallas as pl
from jax.experimental.pallas import tpu as pltpu
from jax.experimental.pallas import tpu_sc as plsc

NC, NS, L = 2, 16, 16
NW = NC * NS
N = 16384
F = 256
ROWS = 64
B_W = N // NW
B_CHUNK = 64
N_CHUNKS = B_W // B_CHUNK


def _emb_body(boxes_hbm, w_hbm, out_hbm, table_v, boxes_v, out_v):
    wid = lax.axis_index("s") * NC + lax.axis_index("c")
    pltpu.sync_copy(w_hbm, table_v)
    lane = lax.iota(jnp.int32, L)

    def chunk_body(ci, carry):
        base = wid * B_W + ci * B_CHUNK
        pltpu.sync_copy(boxes_hbm.at[pl.ds(base, B_CHUNK)], boxes_v)
        for b in range(B_CHUNK // L):
            row = b * L + lane
            rl = []
            rr = []
            lws = []
            rws = []
            for j in range(4):
                bx = plsc.load_gather(
                    boxes_v, [row, jnp.full((L,), j, jnp.int32)])
                data = bx * 16.0
                li = jnp.clip(data, 0.0, 15.0).astype(jnp.int32)
                ri = jnp.minimum(li + 1, 15)
                lw = data - li.astype(jnp.float32)
                rl.append(li * 4 + j)
                rr.append(ri * 4 + j)
                lws.append(lw)
                rws.append(1.0 - lw)

            def f_body(f, carry2, row=row, rl=rl, rr=rr, lws=lws, rws=rws):
                colf = jnp.full((L,), f, jnp.int32)
                for j in range(4):
                    gl = plsc.load_gather(table_v, [rl[j], colf])
                    gr = plsc.load_gather(table_v, [rr[j], colf])
                    res = rws[j] * gl + lws[j] * gr
                    plsc.store_scatter(out_v, [row, colf + (j * F)], res)
                return carry2

            lax.fori_loop(0, F, f_body, 0)
        pltpu.sync_copy(out_v, out_hbm.at[pl.ds(base, B_CHUNK)])
        return carry

    lax.fori_loop(0, N_CHUNKS, chunk_body, 0)


_emb_call = functools.partial(
    pl.kernel,
    out_type=jax.ShapeDtypeStruct((N, 4 * F), jnp.float32),
    mesh=plsc.VectorSubcoreMesh(core_axis_name="c", subcore_axis_name="s"),
    compiler_params=pltpu.CompilerParams(needs_layout_passes=False),
    scratch_types=[
        pltpu.VMEM((ROWS, F), jnp.float32),
        pltpu.VMEM((B_CHUNK, 4), jnp.float32),
        pltpu.VMEM((B_CHUNK, 4 * F), jnp.float32),
    ],
)(_emb_body)


@jax.jit
def kernel(boxes, pos_weight):
    return _emb_call(boxes, pos_weight)


# speedup vs baseline: 2.6297x; 1.6801x over previous
import functools

import jax
import jax.numpy as jnp
from jax import lax
from jax.experimental import pallas as pl
from jax.experimental.pallas import tpu as pltpu
from jax.experimental.pallas import tpu_sc as plsc

NC, NS, L = 2, 16, 16
NW = NC * NS
N = 16384
F = 256
ROWS = 64
D = 4 * F
B_W = N // NW
B_CHUNK = 64
N_CHUNKS = B_W // B_CHUNK


def _emb_body(boxes_hbm, w_hbm, out_hbm, table_v, boxes_v, out_v):
    wid = lax.axis_index("s") * NC + lax.axis_index("c")
    pltpu.sync_copy(w_hbm, table_v)
    lane = lax.iota(jnp.int32, L)
    lane4 = lane * 4
    lane_d = lane * D

    def chunk_body(ci, carry):
        base = wid * B_W + ci * B_CHUNK
        pltpu.sync_copy(boxes_hbm.at[pl.ds(base * 4, B_CHUNK * 4)], boxes_v)
        for b in range(B_CHUNK // L):
            rl = []
            rr = []
            ob = []
            lws = []
            rws = []
            for j in range(4):
                bx = plsc.load_gather(boxes_v, [lane4 + (b * 4 * L + j)])
                data = bx * 16.0
                li = jnp.clip(data, 0.0, 15.0).astype(jnp.int32)
                ri = jnp.minimum(li + 1, 15)
                lw = data - li.astype(jnp.float32)
                rl.append(li * D + (j * F))
                rr.append(ri * D + (j * F))
                ob.append(lane_d + (b * L * D + j * F))
                lws.append(lw)
                rws.append(1.0 - lw)

            @plsc.parallel_loop(0, F, unroll=8)
            def f_body(f, rl=rl, rr=rr, ob=ob, lws=lws, rws=rws):
                fs = jnp.full((L,), f, jnp.int32)
                for j in range(4):
                    gl = plsc.load_gather(table_v, [rl[j] + fs])
                    gr = plsc.load_gather(table_v, [rr[j] + fs])
                    res = rws[j] * gl + lws[j] * gr
                    plsc.store_scatter(out_v, [ob[j] + fs], res)

        pltpu.sync_copy(out_v, out_hbm.at[pl.ds(base * D, B_CHUNK * D)])
        return carry

    lax.fori_loop(0, N_CHUNKS, chunk_body, 0)


_emb_call = functools.partial(
    pl.kernel,
    out_type=jax.ShapeDtypeStruct((N * D,), jnp.float32),
    mesh=plsc.VectorSubcoreMesh(core_axis_name="c", subcore_axis_name="s"),
    compiler_params=pltpu.CompilerParams(
        needs_layout_passes=False, disable_bounds_checks=True),
    scratch_types=[
        pltpu.VMEM((ROWS * F,), jnp.float32),
        pltpu.VMEM((B_CHUNK * 4,), jnp.float32),
        pltpu.VMEM((B_CHUNK * D,), jnp.float32),
    ],
)(_emb_body)


@jax.jit
def kernel(boxes, pos_weight):
    out = _emb_call(boxes.reshape(-1), pos_weight.reshape(-1))
    return out.reshape(N, D)


# speedup vs baseline: 8.8129x; 3.3513x over previous
import functools

import jax
import jax.numpy as jnp
from jax import lax
from jax.experimental import pallas as pl
from jax.experimental.pallas import tpu as pltpu
from jax.experimental.pallas import tpu_sc as plsc

NC, NS, L = 2, 16, 16
NW = NC * NS
N = 16384
F = 256
ROWS = 64
D = 4 * F
B_W = N // NW
B_CHUNK = 64
N_CHUNKS = B_W // B_CHUNK


def _emb_body(boxes_hbm, w_hbm, out_hbm, table_v, boxes_v, out_v,
              loff_v, roff_v, lw_v, rw_v):
    wid = lax.axis_index("s") * NC + lax.axis_index("c")
    pltpu.sync_copy(w_hbm, table_v)
    lane = lax.iota(jnp.int32, L)
    lane4 = lane * 4

    def chunk_body(ci, carry):
        base = wid * B_W + ci * B_CHUNK
        pltpu.sync_copy(boxes_hbm.at[pl.ds(base * 4, B_CHUNK * 4)], boxes_v)
        for b in range(B_CHUNK // L):
            for j in range(4):
                pos = lane4 + (b * 4 * L + j)
                bx = plsc.load_gather(boxes_v, [pos])
                data = bx * 16.0
                li = jnp.clip(data, 0.0, 15.0).astype(jnp.int32)
                ri = jnp.minimum(li + 1, 15)
                lw = data - li.astype(jnp.float32)
                plsc.store_scatter(loff_v, [pos], li * D + (j * F))
                plsc.store_scatter(roff_v, [pos], ri * D + (j * F))
                plsc.store_scatter(lw_v, [pos], lw)
                plsc.store_scatter(rw_v, [pos], 1.0 - lw)

        @plsc.parallel_loop(0, B_CHUNK, unroll=2)
        def n_body(n):
            k = n * 4
            lv = loff_v[pl.ds(k, L)]
            rv = roff_v[pl.ds(k, L)]
            lwv4 = lw_v[pl.ds(k, L)]
            rwv4 = rw_v[pl.ds(k, L)]
            obase = n * D
            for j in range(4):
                lo = lv[j]
                ro = rv[j]
                lwv = jnp.full((L,), lwv4[j], jnp.float32)
                rwv = jnp.full((L,), rwv4[j], jnp.float32)
                ob = obase + j * F
                for t in range(F // L):
                    gl = table_v[pl.ds(lo + t * L, L)]
                    gr = table_v[pl.ds(ro + t * L, L)]
                    out_v[pl.ds(ob + t * L, L)] = rwv * gl + lwv * gr

        pltpu.sync_copy(out_v, out_hbm.at[pl.ds(base * D, B_CHUNK * D)])
        return carry

    lax.fori_loop(0, N_CHUNKS, chunk_body, 0)


_emb_call = functools.partial(
    pl.kernel,
    out_type=jax.ShapeDtypeStruct((N * D,), jnp.float32),
    mesh=plsc.VectorSubcoreMesh(core_axis_name="c", subcore_axis_name="s"),
    compiler_params=pltpu.CompilerParams(
        needs_layout_passes=False, disable_bounds_checks=True),
    scratch_types=[
        pltpu.VMEM((ROWS * F,), jnp.float32),
        pltpu.VMEM((B_CHUNK * 4,), jnp.float32),
        pltpu.VMEM((B_CHUNK * D,), jnp.float32),
        pltpu.VMEM((4 * B_CHUNK,), jnp.int32),
        pltpu.VMEM((4 * B_CHUNK,), jnp.int32),
        pltpu.VMEM((4 * B_CHUNK,), jnp.float32),
        pltpu.VMEM((4 * B_CHUNK,), jnp.float32),
    ],
)(_emb_body)


@jax.jit
def kernel(boxes, pos_weight):
    out = _emb_call(boxes.reshape(-1), pos_weight.reshape(-1))
    return out.reshape(N, D)


# speedup vs baseline: 8.9567x; 1.0163x over previous
import functools

import jax
import jax.numpy as jnp
from jax import lax
from jax.experimental import pallas as pl
from jax.experimental.pallas import tpu as pltpu
from jax.experimental.pallas import tpu_sc as plsc

NC, NS, L = 2, 16, 16
NW = NC * NS
N = 16384
F = 256
ROWS = 64
D = 4 * F
B_W = N // NW
B_HALF = 32
N_PAIRS = B_W // (2 * B_HALF)


def _emb_body(boxes_hbm, w_hbm, out_hbm, table_v, boxes_v,
              out0_v, out1_v, loff_v, roff_v, lw_v, rw_v, sem0, sem1):
    wid = lax.axis_index("s") * NC + lax.axis_index("c")
    pltpu.sync_copy(w_hbm, table_v)
    pltpu.sync_copy(boxes_hbm.at[pl.ds(wid * (B_W * 4), B_W * 4)], boxes_v)
    lane = lax.iota(jnp.int32, L)
    lane4 = lane * 4

    def stage_body(blk, carry):
        for j in range(4):
            pos = lane4 + (blk * 4 * L + j)
            bx = plsc.load_gather(boxes_v, [pos])
            data = bx * 16.0
            li = jnp.clip(data, 0.0, 15.0).astype(jnp.int32)
            ri = jnp.minimum(li + 1, 15)
            lw = data - li.astype(jnp.float32)
            plsc.store_scatter(loff_v, [pos], li * D + (j * F))
            plsc.store_scatter(roff_v, [pos], ri * D + (j * F))
            plsc.store_scatter(lw_v, [pos], lw)
            plsc.store_scatter(rw_v, [pos], 1.0 - lw)
        return carry

    lax.fori_loop(0, B_W // L, stage_body, 0)

    bufs = (out0_v, out1_v)
    sems = (sem0, sem1)

    def pair_body(ci, carry):
        for h in range(2):
            buf, sem = bufs[h], sems[h]
            idx = ci * 2 + h

            @pl.when(ci > 0)
            def _wait_prev():
                pltpu.make_async_copy(
                    buf, out_hbm.at[pl.ds(0, B_HALF * D)], sem).wait()

            kbase = idx * B_HALF * 4

            @plsc.parallel_loop(0, B_HALF, unroll=2)
            def n_body(n):
                k = kbase + n * 4
                lv = loff_v[pl.ds(k, L)]
                rv = roff_v[pl.ds(k, L)]
                lwv4 = lw_v[pl.ds(k, L)]
                rwv4 = rw_v[pl.ds(k, L)]
                obase = n * D
                for j in range(4):
                    lo = lv[j]
                    ro = rv[j]
                    lwv = jnp.full((L,), lwv4[j], jnp.float32)
                    rwv = jnp.full((L,), rwv4[j], jnp.float32)
                    ob = obase + j * F
                    for t in range(F // L):
                        gl = table_v[pl.ds(lo + t * L, L)]
                        gr = table_v[pl.ds(ro + t * L, L)]
                        out = rwv * gl + lwv * gr
                        bufs[h][pl.ds(ob + t * L, L)] = out

            hbase = (wid * B_W + idx * B_HALF) * D
            pltpu.async_copy(buf, out_hbm.at[pl.ds(hbase, B_HALF * D)], sem)
        return carry

    lax.fori_loop(0, N_PAIRS, pair_body, 0)
    for h in range(2):
        pltpu.make_async_copy(
            bufs[h], out_hbm.at[pl.ds(0, B_HALF * D)], sems[h]).wait()


_emb_call = functools.partial(
    pl.kernel,
    out_type=jax.ShapeDtypeStruct((N * D,), jnp.float32),
    mesh=plsc.VectorSubcoreMesh(core_axis_name="c", subcore_axis_name="s"),
    compiler_params=pltpu.CompilerParams(
        needs_layout_passes=False, disable_bounds_checks=True),
    scratch_types=[
        pltpu.VMEM((ROWS * F,), jnp.float32),
        pltpu.VMEM((B_W * 4,), jnp.float32),
        pltpu.VMEM((B_HALF * D,), jnp.float32),
        pltpu.VMEM((B_HALF * D,), jnp.float32),
        pltpu.VMEM((4 * B_W,), jnp.int32),
        pltpu.VMEM((4 * B_W,), jnp.int32),
        pltpu.VMEM((4 * B_W,), jnp.float32),
        pltpu.VMEM((4 * B_W,), jnp.float32),
        pltpu.SemaphoreType.DMA,
        pltpu.SemaphoreType.DMA,
    ],
)(_emb_body)


@jax.jit
def kernel(boxes, pos_weight):
    out = _emb_call(boxes.reshape(-1), pos_weight.reshape(-1))
    return out.reshape(N, D)
